# baseline (speedup 1.0000x reference)
import jax
import jax.numpy as jnp
from jax.experimental import pallas as pl
from jax.experimental.pallas import tpu as pltpu

M = 65
N = 100000
F = 128
NUM_HEAD = 8
SLOPE = 0.2
BLK = 2048
NBLK = (N + BLK - 1) // BLK
NOPS = 1
STEPS = (NBLK + NOPS - 1) // NOPS
INT_MAX = jnp.iinfo(jnp.int32).max


def _scan_body(*refs):
    adj_refs = refs[:NOPS]
    blk_out_ref, col48_out_ref, max_sc, blk_sc = refs[NOPS:]
    j = pl.program_id(0)

    @pl.when(j == 0)
    def _init():
        max_sc[...] = jnp.full((M, 1), -1.0, jnp.float32)
        blk_sc[...] = jnp.zeros((M, 1), jnp.int32)

    for k, ref in enumerate(adj_refs):
        bid = jnp.minimum(k * STEPS + j, NBLK - 1)
        vals = ref[...]
        if k == NOPS - 1:
            cols = bid * BLK + jax.lax.broadcasted_iota(
                jnp.int32, (M, BLK), 1)
            vals = jnp.where(cols < N, vals, -1.0)
        bmax = jnp.max(vals, axis=1, keepdims=True)
        better = (bmax > max_sc[...]) | (
            (bmax == max_sc[...]) & (bid < blk_sc[...]))
        max_sc[...] = jnp.where(better, bmax, max_sc[...])
        blk_sc[...] = jnp.where(better, bid, blk_sc[...])
        if k == NOPS - 1:
            @pl.when(j == STEPS - 1)
            def _fin():
                col48_out_ref[...] = jnp.min(
                    jnp.where(vals == bmax, cols, INT_MAX),
                    axis=1, keepdims=True)
                blk_out_ref[...] = blk_sc[...]


def _find_body(blk_smem, adj_ref, blk_v_ref, col48_v_ref, idx_out_ref,
               win_sc, sem):
    def issue(r, _):
        start = jnp.minimum(blk_smem[r], NBLK - 2) * BLK
        pltpu.make_async_copy(
            adj_ref.at[pl.ds(r, 1), :, pl.ds(start, BLK)],
            win_sc.at[pl.ds(r, 1)], sem).start()
        return 0
    jax.lax.fori_loop(0, M, issue, 0)

    def drain(r, _):
        pltpu.make_async_copy(
            adj_ref.at[pl.ds(0, 1), :, pl.ds(0, BLK)],
            win_sc.at[pl.ds(r, 1)], sem).wait()
        return 0
    jax.lax.fori_loop(0, M, drain, 0)

    w2 = win_sc[...].reshape(M, BLK)
    starts = jnp.minimum(blk_v_ref[...], NBLK - 2) * BLK
    cols = starts + jax.lax.broadcasted_iota(jnp.int32, (M, BLK), 1)
    m = jnp.max(w2, axis=1, keepdims=True)
    idxw = jnp.min(jnp.where(w2 == m, cols, INT_MAX), axis=1, keepdims=True)
    idx_out_ref[...] = jnp.where(
        blk_v_ref[...] == NBLK - 1, col48_v_ref[...], idxw)


def _gat_body(idx_ref, x_ref, w_ref, a_ref, b_ref, out_ref, rows_sc, sem):
    def issue(i, _):
        pltpu.make_async_copy(
            x_ref.at[pl.ds(idx_ref[i], 1), :],
            rows_sc.at[pl.ds(i, 1), :], sem).start()
        return 0
    jax.lax.fori_loop(0, M, issue, 0)

    def drain(i, _):
        pltpu.make_async_copy(
            x_ref.at[pl.ds(0, 1), :],
            rows_sc.at[pl.ds(i, 1), :], sem).wait()
        return 0
    jax.lax.fori_loop(0, M, drain, 0)

    sel = rows_sc[...]
    h = jnp.dot(sel, w_ref[...], preferred_element_type=jnp.float32)
    a0 = a_ref[0:1, :]
    a1 = a_ref[1:2, :]
    c = jnp.sum(h[0:1, :] * a0)
    d = jnp.sum(h * a1, axis=1, keepdims=True)
    lg = c + d
    lg = jnp.where(lg >= 0, lg, SLOPE * lg)
    ridx = jax.lax.broadcasted_iota(jnp.int32, (M, 1), 0)
    e = jnp.where(ridx >= 1, jnp.exp(lg), 0.0)
    alpha = e / jnp.sum(e)
    hp = jnp.sum(alpha * h, axis=0, keepdims=True) + b_ref[...]
    out_ref[...] = jnp.broadcast_to(hp, (NUM_HEAD, F))


def _adj_spec(k):
    return pl.BlockSpec(
        (M, BLK), lambda j, k=k: (0, jnp.minimum(k * STEPS + j, NBLK - 1)))


def kernel(x, adj, weight, a, bias):
    blk2, col48 = pl.pallas_call(
        _scan_body,
        grid=(STEPS,),
        in_specs=[_adj_spec(k) for k in range(NOPS)],
        out_specs=(pl.BlockSpec((M, 1), lambda j: (0, 0)),
                   pl.BlockSpec((M, 1), lambda j: (0, 0))),
        out_shape=(jax.ShapeDtypeStruct((M, 1), jnp.int32),
                   jax.ShapeDtypeStruct((M, 1), jnp.int32)),
        scratch_shapes=[pltpu.VMEM((M, 1), jnp.float32),
                        pltpu.VMEM((M, 1), jnp.int32)],
    )(*([adj] * NOPS))
    blk = blk2.reshape(M)

    idx2 = pl.pallas_call(
        _find_body,
        grid_spec=pltpu.PrefetchScalarGridSpec(
            num_scalar_prefetch=1,
            grid=(1,),
            in_specs=[
                pl.BlockSpec(memory_space=pl.ANY),
                pl.BlockSpec((M, 1), lambda i, b: (0, 0)),
                pl.BlockSpec((M, 1), lambda i, b: (0, 0)),
            ],
            out_specs=pl.BlockSpec((M, 1), lambda i, b: (0, 0)),
            scratch_shapes=[pltpu.VMEM((M, 1, BLK), jnp.float32),
                            pltpu.SemaphoreType.DMA],
        ),
        out_shape=jax.ShapeDtypeStruct((M, 1), jnp.int32),
    )(blk, adj.reshape(M, 1, N), blk2, col48)
    idx = idx2.reshape(M)

    out = pl.pallas_call(
        _gat_body,
        grid_spec=pltpu.PrefetchScalarGridSpec(
            num_scalar_prefetch=1,
            grid=(1,),
            in_specs=[
                pl.BlockSpec(memory_space=pl.ANY),
                pl.BlockSpec((F, F), lambda i, idx_ref: (0, 0)),
                pl.BlockSpec((2, F), lambda i, idx_ref: (0, 0)),
                pl.BlockSpec((1, F), lambda i, idx_ref: (0, 0)),
            ],
            out_specs=pl.BlockSpec((NUM_HEAD, F), lambda i, idx_ref: (0, 0)),
            scratch_shapes=[pltpu.VMEM((M, F), jnp.float32),
                            pltpu.SemaphoreType.DMA],
        ),
        out_shape=jax.ShapeDtypeStruct((NUM_HEAD, F), jnp.float32),
    )(idx, x, weight, a.reshape(2, F), bias.reshape(1, F))
    return out.reshape(NUM_HEAD * F)


# speedup vs baseline: 1.3508x; 1.3508x over previous
import jax
import jax.numpy as jnp
from jax.experimental import pallas as pl
from jax.experimental.pallas import tpu as pltpu

M = 65
N = 100000
F = 128
NUM_HEAD = 8
SLOPE = 0.2
BLK = 12544
NBLK = (N + BLK - 1) // BLK
NOPS = 1
STEPS = (NBLK + NOPS - 1) // NOPS
INT_MAX = jnp.iinfo(jnp.int32).max


def _scan_body(*refs):
    adj_refs = refs[:NOPS]
    blk_out_ref, col48_out_ref, max_sc, blk_sc = refs[NOPS:]
    j = pl.program_id(0)

    @pl.when(j == 0)
    def _init():
        max_sc[...] = jnp.full((M, 1), -1.0, jnp.float32)
        blk_sc[...] = jnp.zeros((M, 1), jnp.int32)

    for k, ref in enumerate(adj_refs):
        bid = jnp.minimum(k * STEPS + j, NBLK - 1)
        vals = ref[...]
        if k == NOPS - 1:
            cols = bid * BLK + jax.lax.broadcasted_iota(
                jnp.int32, (M, BLK), 1)
            vals = jnp.where(cols < N, vals, -1.0)
        bmax = jnp.max(vals, axis=1, keepdims=True)
        better = (bmax > max_sc[...]) | (
            (bmax == max_sc[...]) & (bid < blk_sc[...]))
        max_sc[...] = jnp.where(better, bmax, max_sc[...])
        blk_sc[...] = jnp.where(better, bid, blk_sc[...])
        if k == NOPS - 1:
            @pl.when(j == STEPS - 1)
            def _fin():
                col48_out_ref[...] = jnp.min(
                    jnp.where(vals == bmax, cols, INT_MAX),
                    axis=1, keepdims=True)
                blk_out_ref[...] = blk_sc[...]


def _find_body(blk_smem, adj_ref, blk_v_ref, col48_v_ref, idx_out_ref,
               win_sc, sem):
    def issue(r, _):
        start = jnp.minimum(blk_smem[r], NBLK - 2) * BLK
        pltpu.make_async_copy(
            adj_ref.at[pl.ds(r, 1), :, pl.ds(start, BLK)],
            win_sc.at[pl.ds(r, 1)], sem).start()
        return 0
    jax.lax.fori_loop(0, M, issue, 0)

    def drain(r, _):
        pltpu.make_async_copy(
            adj_ref.at[pl.ds(0, 1), :, pl.ds(0, BLK)],
            win_sc.at[pl.ds(r, 1)], sem).wait()
        return 0
    jax.lax.fori_loop(0, M, drain, 0)

    w2 = win_sc[...].reshape(M, BLK)
    starts = jnp.minimum(blk_v_ref[...], NBLK - 2) * BLK
    cols = starts + jax.lax.broadcasted_iota(jnp.int32, (M, BLK), 1)
    m = jnp.max(w2, axis=1, keepdims=True)
    idxw = jnp.min(jnp.where(w2 == m, cols, INT_MAX), axis=1, keepdims=True)
    idx_out_ref[...] = jnp.where(
        blk_v_ref[...] == NBLK - 1, col48_v_ref[...], idxw)


def _gat_body(idx_ref, x_ref, w_ref, a_ref, b_ref, out_ref, rows_sc, sem):
    def issue(i, _):
        pltpu.make_async_copy(
            x_ref.at[pl.ds(idx_ref[i], 1), :],
            rows_sc.at[pl.ds(i, 1), :], sem).start()
        return 0
    jax.lax.fori_loop(0, M, issue, 0)

    def drain(i, _):
        pltpu.make_async_copy(
            x_ref.at[pl.ds(0, 1), :],
            rows_sc.at[pl.ds(i, 1), :], sem).wait()
        return 0
    jax.lax.fori_loop(0, M, drain, 0)

    sel = rows_sc[...]
    h = jnp.dot(sel, w_ref[...], preferred_element_type=jnp.float32)
    a0 = a_ref[0:1, :]
    a1 = a_ref[1:2, :]
    c = jnp.sum(h[0:1, :] * a0)
    d = jnp.sum(h * a1, axis=1, keepdims=True)
    lg = c + d
    lg = jnp.where(lg >= 0, lg, SLOPE * lg)
    ridx = jax.lax.broadcasted_iota(jnp.int32, (M, 1), 0)
    e = jnp.where(ridx >= 1, jnp.exp(lg), 0.0)
    alpha = e / jnp.sum(e)
    hp = jnp.sum(alpha * h, axis=0, keepdims=True) + b_ref[...]
    out_ref[...] = jnp.broadcast_to(hp, (NUM_HEAD, F))


def _adj_spec(k):
    return pl.BlockSpec(
        (M, BLK), lambda j, k=k: (0, jnp.minimum(k * STEPS + j, NBLK - 1)))


def kernel(x, adj, weight, a, bias):
    blk2, col48 = pl.pallas_call(
        _scan_body,
        grid=(STEPS,),
        in_specs=[_adj_spec(k) for k in range(NOPS)],
        out_specs=(pl.BlockSpec((M, 1), lambda j: (0, 0)),
                   pl.BlockSpec((M, 1), lambda j: (0, 0))),
        out_shape=(jax.ShapeDtypeStruct((M, 1), jnp.int32),
                   jax.ShapeDtypeStruct((M, 1), jnp.int32)),
        scratch_shapes=[pltpu.VMEM((M, 1), jnp.float32),
                        pltpu.VMEM((M, 1), jnp.int32)],
    )(*([adj] * NOPS))
    blk = blk2.reshape(M)

    idx2 = pl.pallas_call(
        _find_body,
        grid_spec=pltpu.PrefetchScalarGridSpec(
            num_scalar_prefetch=1,
            grid=(1,),
            in_specs=[
                pl.BlockSpec(memory_space=pl.ANY),
                pl.BlockSpec((M, 1), lambda i, b: (0, 0)),
                pl.BlockSpec((M, 1), lambda i, b: (0, 0)),
            ],
            out_specs=pl.BlockSpec((M, 1), lambda i, b: (0, 0)),
            scratch_shapes=[pltpu.VMEM((M, 1, BLK), jnp.float32),
                            pltpu.SemaphoreType.DMA],
        ),
        out_shape=jax.ShapeDtypeStruct((M, 1), jnp.int32),
    )(blk, adj.reshape(M, 1, N), blk2, col48)
    idx = idx2.reshape(M)

    out = pl.pallas_call(
        _gat_body,
        grid_spec=pltpu.PrefetchScalarGridSpec(
            num_scalar_prefetch=1,
            grid=(1,),
            in_specs=[
                pl.BlockSpec(memory_space=pl.ANY),
                pl.BlockSpec((F, F), lambda i, idx_ref: (0, 0)),
                pl.BlockSpec((2, F), lambda i, idx_ref: (0, 0)),
                pl.BlockSpec((1, F), lambda i, idx_ref: (0, 0)),
            ],
            out_specs=pl.BlockSpec((NUM_HEAD, F), lambda i, idx_ref: (0, 0)),
            scratch_shapes=[pltpu.VMEM((M, F), jnp.float32),
                            pltpu.SemaphoreType.DMA],
        ),
        out_shape=jax.ShapeDtypeStruct((NUM_HEAD, F), jnp.float32),
    )(idx, x, weight, a.reshape(2, F), bias.reshape(1, F))
    return out.reshape(NUM_HEAD * F)


# speedup vs baseline: 3.2954x; 2.4397x over previous
import jax
import jax.numpy as jnp
from jax.experimental import pallas as pl
from jax.experimental.pallas import tpu as pltpu

M = 65
N = 100000
F = 128
NUM_HEAD = 8
SLOPE = 0.2
BLK = 12544
NBLK = (N + BLK - 1) // BLK
INT_MAX = jnp.iinfo(jnp.int32).max


def _scan_body(adj_ref, idx_out_ref, max_sc, idx_sc):
    j = pl.program_id(0)

    @pl.when(j == 0)
    def _init():
        max_sc[...] = jnp.full((M, 1), -1.0, jnp.float32)
        idx_sc[...] = jnp.zeros((M, 1), jnp.int32)

    cols = j * BLK + jax.lax.broadcasted_iota(jnp.int32, (M, BLK), 1)
    vals = jnp.where(cols < N, adj_ref[...], -1.0)
    bmax = jnp.max(vals, axis=1, keepdims=True)
    bidx = jnp.min(jnp.where(vals == bmax, cols, INT_MAX),
                   axis=1, keepdims=True)
    better = bmax > max_sc[...]
    max_sc[...] = jnp.where(better, bmax, max_sc[...])
    idx_sc[...] = jnp.where(better, bidx, idx_sc[...])

    @pl.when(j == NBLK - 1)
    def _fin():
        idx_out_ref[...] = idx_sc[...]


def _gat_body(idx_ref, x_ref, w_ref, a_ref, b_ref, out_ref, rows_sc, sem):
    def issue(i, _):
        pltpu.make_async_copy(
            x_ref.at[pl.ds(idx_ref[i], 1), :],
            rows_sc.at[pl.ds(i, 1), :], sem).start()
        return 0
    jax.lax.fori_loop(0, M, issue, 0)

    def drain(i, _):
        pltpu.make_async_copy(
            x_ref.at[pl.ds(0, 1), :],
            rows_sc.at[pl.ds(i, 1), :], sem).wait()
        return 0
    jax.lax.fori_loop(0, M, drain, 0)

    sel = rows_sc[...]
    h = jnp.dot(sel, w_ref[...], preferred_element_type=jnp.float32)
    a0 = a_ref[0:1, :]
    a1 = a_ref[1:2, :]
    c = jnp.sum(h[0:1, :] * a0)
    d = jnp.sum(h * a1, axis=1, keepdims=True)
    lg = c + d
    lg = jnp.where(lg >= 0, lg, SLOPE * lg)
    ridx = jax.lax.broadcasted_iota(jnp.int32, (M, 1), 0)
    e = jnp.where(ridx >= 1, jnp.exp(lg), 0.0)
    alpha = e / jnp.sum(e)
    hp = jnp.sum(alpha * h, axis=0, keepdims=True) + b_ref[...]
    out_ref[...] = jnp.broadcast_to(hp, (NUM_HEAD, F))


def kernel(x, adj, weight, a, bias):
    idx2 = pl.pallas_call(
        _scan_body,
        grid=(NBLK,),
        in_specs=[pl.BlockSpec((M, BLK), lambda j: (0, j))],
        out_specs=pl.BlockSpec((M, 1), lambda j: (0, 0)),
        out_shape=jax.ShapeDtypeStruct((M, 1), jnp.int32),
        scratch_shapes=[pltpu.VMEM((M, 1), jnp.float32),
                        pltpu.VMEM((M, 1), jnp.int32)],
    )(adj)
    idx = idx2.reshape(M)

    out = pl.pallas_call(
        _gat_body,
        grid_spec=pltpu.PrefetchScalarGridSpec(
            num_scalar_prefetch=1,
            grid=(1,),
            in_specs=[
                pl.BlockSpec(memory_space=pl.ANY),
                pl.BlockSpec((F, F), lambda i, idx_ref: (0, 0)),
                pl.BlockSpec((2, F), lambda i, idx_ref: (0, 0)),
                pl.BlockSpec((1, F), lambda i, idx_ref: (0, 0)),
            ],
            out_specs=pl.BlockSpec((NUM_HEAD, F), lambda i, idx_ref: (0, 0)),
            scratch_shapes=[pltpu.VMEM((M, F), jnp.float32),
                            pltpu.SemaphoreType.DMA],
        ),
        out_shape=jax.ShapeDtypeStruct((NUM_HEAD, F), jnp.float32),
    )(idx, x, weight, a.reshape(2, F), bias.reshape(1, F))
    return out.reshape(NUM_HEAD * F)


# speedup vs baseline: 3.5118x; 1.0657x over previous
import jax
import jax.numpy as jnp
from jax.experimental import pallas as pl
from jax.experimental.pallas import tpu as pltpu

M = 65
N = 100000
F = 128
NUM_HEAD = 8
SLOPE = 0.2
BLK = 25088
NBLK = (N + BLK - 1) // BLK
INT_MAX = jnp.iinfo(jnp.int32).max


def _scan_body(adj_ref, idx_out_ref, max_sc, idx_sc):
    j = pl.program_id(0)

    @pl.when(j == 0)
    def _init():
        max_sc[...] = jnp.full((M, 1), -1.0, jnp.float32)
        idx_sc[...] = jnp.zeros((M, 1), jnp.int32)

    cols = j * BLK + jax.lax.broadcasted_iota(jnp.int32, (M, BLK), 1)
    vals = jnp.where(cols < N, adj_ref[...], -1.0)
    bmax = jnp.max(vals, axis=1, keepdims=True)
    bidx = jnp.min(jnp.where(vals == bmax, cols, INT_MAX),
                   axis=1, keepdims=True)
    better = bmax > max_sc[...]
    max_sc[...] = jnp.where(better, bmax, max_sc[...])
    idx_sc[...] = jnp.where(better, bidx, idx_sc[...])

    @pl.when(j == NBLK - 1)
    def _fin():
        idx_out_ref[...] = idx_sc[...]


def _gat_body(idx_ref, x_ref, w_ref, a_ref, b_ref, out_ref, rows_sc, sem):
    for i in range(M):
        pltpu.make_async_copy(
            x_ref.at[pl.ds(idx_ref[i], 1), :],
            rows_sc.at[pl.ds(i, 1), :], sem).start()
    for i in range(M):
        pltpu.make_async_copy(
            x_ref.at[pl.ds(0, 1), :],
            rows_sc.at[pl.ds(i, 1), :], sem).wait()

    sel = rows_sc[...]
    h = jnp.dot(sel, w_ref[...], preferred_element_type=jnp.float32)
    a0 = a_ref[0:1, :]
    a1 = a_ref[1:2, :]
    c = jnp.sum(h[0:1, :] * a0)
    d = jnp.sum(h * a1, axis=1, keepdims=True)
    lg = c + d
    lg = jnp.where(lg >= 0, lg, SLOPE * lg)
    ridx = jax.lax.broadcasted_iota(jnp.int32, (M, 1), 0)
    e = jnp.where(ridx >= 1, jnp.exp(lg), 0.0)
    alpha = e / jnp.sum(e)
    hp = jnp.sum(alpha * h, axis=0, keepdims=True) + b_ref[...]
    out_ref[...] = jnp.broadcast_to(hp, (NUM_HEAD, F))


def kernel(x, adj, weight, a, bias):
    idx2 = pl.pallas_call(
        _scan_body,
        grid=(NBLK,),
        in_specs=[pl.BlockSpec((M, BLK), lambda j: (0, j))],
        out_specs=pl.BlockSpec((M, 1), lambda j: (0, 0)),
        out_shape=jax.ShapeDtypeStruct((M, 1), jnp.int32),
        scratch_shapes=[pltpu.VMEM((M, 1), jnp.float32),
                        pltpu.VMEM((M, 1), jnp.int32)],
    )(adj)
    idx = idx2.reshape(M)

    out = pl.pallas_call(
        _gat_body,
        grid_spec=pltpu.PrefetchScalarGridSpec(
            num_scalar_prefetch=1,
            grid=(1,),
            in_specs=[
                pl.BlockSpec(memory_space=pl.ANY),
                pl.BlockSpec((F, F), lambda i, idx_ref: (0, 0)),
                pl.BlockSpec((2, F), lambda i, idx_ref: (0, 0)),
                pl.BlockSpec((1, F), lambda i, idx_ref: (0, 0)),
            ],
            out_specs=pl.BlockSpec((NUM_HEAD, F), lambda i, idx_ref: (0, 0)),
            scratch_shapes=[pltpu.VMEM((M, F), jnp.float32),
                            pltpu.SemaphoreType.DMA],
        ),
        out_shape=jax.ShapeDtypeStruct((NUM_HEAD, F), jnp.float32),
    )(idx, x, weight, a.reshape(2, F), bias.reshape(1, F))
    return out.reshape(NUM_HEAD * F)
